# baseline (speedup 1.0000x reference)
import numpy as np
import jax
import jax.numpy as jnp
from jax import lax
from jax.experimental import pallas as pl
from jax.experimental.pallas import tpu as pltpu


def _edge_tile(e_pad):
    for te in (512, 256, 128):
        if e_pad % te == 0:
            return te
    return e_pad


def _conv1_extract_kernel(ea_ref, s_ref, m_ref, x_ref,
                          w1a_ref, b1a_ref, w1b_ref, b1b_ref,
                          wr1_ref, bc1_ref, bs_ref, bd_ref,
                          h1_ref, idx_ref, acc_ref):
    t = pl.program_id(0)
    f32 = jnp.float32

    @pl.when(t == 0)
    def _init():
        acc_ref[...] = jnp.zeros_like(acc_ref)

    ea = ea_ref[...]
    w1a = w1a_ref[...]
    hid = jnp.maximum(ea[:, 0:1] * w1a[0:1, :] + ea[:, 1:2] * w1a[1:2, :]
                      + b1a_ref[...], 0.0)
    z = jnp.dot(hid, w1b_ref[...], preferred_element_type=f32) + b1b_ref[...]

    s = s_ref[...]
    m = m_ref[...]

    xg = jnp.dot(s, x_ref[...], preferred_element_type=f32)
    msg = xg[:, 0:1] * z[:, 0:32] + xg[:, 1:2] * z[:, 32:64]
    acc_ref[...] += jnp.dot(m, msg, preferred_element_type=f32)

    hp = lax.Precision.HIGHEST
    idx_ref[...] = (jnp.dot(s, bs_ref[...], preferred_element_type=f32,
                            precision=hp)
                    + lax.dot_general(m, bd_ref[...],
                                      (((0,), (0,)), ((), ())),
                                      preferred_element_type=f32,
                                      precision=hp))

    @pl.when(t == pl.num_programs(0) - 1)
    def _finalize():
        x = x_ref[...]
        wr = wr1_ref[...]
        root = x[:, 0:1] * wr[0:1, :] + x[:, 1:2] * wr[1:2, :]
        h1_ref[...] = jnp.maximum(acc_ref[...] + root + bc1_ref[...], 0.0)


def _conv2_head_kernel(ea_ref, idx_ref, hrs_ref,
                       w2a_ref, b2a_ref, w2b_ref, b2b_ref,
                       r2_ref, q2_ref, q2t_ref,
                       wr2b_ref, bc2t_ref, wf1b_ref, bf1t_ref,
                       wf2b_ref, bf2t_ref,
                       out_ref, acc_ref):
    t = pl.program_id(0)
    f32 = jnp.float32
    n_hi = acc_ref.shape[0]
    te = ea_ref.shape[0]

    @pl.when(t == 0)
    def _init():
        acc_ref[...] = jnp.zeros_like(acc_ref)

    ea = ea_ref[...]
    w2a = w2a_ref[...]
    hid = jnp.maximum(ea[:, 0:1] * w2a[0:1, :] + ea[:, 1:2] * w2a[1:2, :]
                      + b2a_ref[...], 0.0)
    z = jnp.dot(hid, w2b_ref[...], preferred_element_type=f32) + b2b_ref[...]

    idx = idx_ref[...]
    src = jnp.round(idx[:, 0:1])
    w = idx[:, 2:3]
    dst = jnp.round(idx[:, 1:2] / jnp.maximum(w, 1e-30))

    shi = jnp.floor(src * (1.0 / 32.0))
    slo = src - 32.0 * shi
    dhi = jnp.floor(dst * (1.0 / 32.0))
    dlo = dst - 32.0 * dhi

    ihi = lax.broadcasted_iota(jnp.int32, (te, n_hi), 1).astype(f32)
    ilo = lax.broadcasted_iota(jnp.int32, (te, 32), 1).astype(f32)
    oh_shi = (shi == ihi).astype(f32)
    oh_slo = (slo == ilo).astype(f32)
    oh_dhi = (dhi == ihi).astype(f32)
    oh_dlo = (dlo == ilo).astype(f32)

    r2 = r2_ref[...]
    q2 = q2_ref[...]
    hrs = hrs_ref[...]

    hb = jnp.dot(oh_shi, hrs, preferred_element_type=f32)
    rep_slo = jnp.dot(oh_slo, r2, preferred_element_type=f32)
    hg = jnp.dot(hb * rep_slo, q2, preferred_element_type=f32)

    hg_rep = jnp.dot(hg, r2, preferred_element_type=f32)
    msg = jnp.dot(hg_rep * z, q2, preferred_element_type=f32)

    msg_t = jnp.dot(w * msg, q2t_ref[...], preferred_element_type=f32)
    rep_dlo = jnp.dot(oh_dlo, r2, preferred_element_type=f32)
    acc_ref[...] += lax.dot_general(oh_dhi, rep_dlo * msg_t,
                                    (((0,), (0,)), ((), ())),
                                    preferred_element_type=f32)

    @pl.when(t == pl.num_programs(0) - 1)
    def _finalize():
        hrs_f = hrs_ref[...]
        h2 = jnp.maximum(acc_ref[...]
                         + jnp.dot(hrs_f, wr2b_ref[...], preferred_element_type=f32)
                         + bc2t_ref[...], 0.0)
        h3 = jnp.maximum(jnp.dot(h2, wf1b_ref[...], preferred_element_type=f32)
                         + bf1t_ref[...], 0.0)
        out_ref[...] = (jnp.dot(h3, wf2b_ref[...], preferred_element_type=f32)
                        + bf2t_ref[...])


def _full(arr):
    nd = arr.ndim
    return pl.BlockSpec(arr.shape, lambda t, _n=nd: (0,) * _n)


def kernel(x, edge_attr_pad, S, M,
           w1a, b1a, w1b, b1b, w2a, b2a, w2b, b2b,
           wr1, bc1, wr2, bc2, wfc1, bfc1, wfc2, bfc2, r2, q2):
    f32 = jnp.float32
    n = x.shape[0]
    e_pad = edge_attr_pad.shape[0]
    te = _edge_tile(e_pad)
    grid = (e_pad // te,)
    cparams = pltpu.CompilerParams(dimension_semantics=("arbitrary",))

    ar = np.arange(n, dtype=np.float32)
    bs = np.zeros((n, 8), np.float32)
    bs[:, 0] = ar
    bd = np.zeros((n, 8), np.float32)
    bd[:, 1] = ar
    bd[:, 2] = 1.0
    jj = np.arange(32 * 32)
    q2t = (jj[None, :] % 32 == np.arange(32)[:, None]).astype(np.float32)

    conv1_args = (edge_attr_pad, S, M, x, w1a, b1a, w1b, b1b, wr1, bc1,
                  jnp.asarray(bs), jnp.asarray(bd))
    h1, idx = pl.pallas_call(
        _conv1_extract_kernel,
        out_shape=[jax.ShapeDtypeStruct((n, 32), f32),
                   jax.ShapeDtypeStruct((e_pad, 8), f32)],
        grid=grid,
        in_specs=[
            pl.BlockSpec((te, 2), lambda t: (t, 0)),
            pl.BlockSpec((te, n), lambda t: (t, 0)),
            pl.BlockSpec((n, te), lambda t: (0, t)),
        ] + [_full(a) for a in conv1_args[3:]],
        out_specs=[pl.BlockSpec((n, 32), lambda t: (0, 0)),
                   pl.BlockSpec((te, 8), lambda t: (t, 0))],
        scratch_shapes=[pltpu.VMEM((n, 32), f32)],
        compiler_params=cparams,
    )(*conv1_args)

    n_hi = n // 32
    h1_rs = h1.reshape(n_hi, 32 * 32)
    eye32 = jnp.eye(32, dtype=f32)
    wr2b = jnp.kron(eye32, wr2)
    wf1b = jnp.kron(eye32, wfc1)
    wf2b = jnp.kron(eye32, wfc2)
    bc2t = jnp.tile(bc2, (1, 32))
    bf1t = jnp.tile(bfc1, (1, 32))
    bf2t = jnp.tile(bfc2, (1, 32))

    conv2_args = (edge_attr_pad, idx, h1_rs, w2a, b2a, w2b, b2b,
                  r2, q2, jnp.asarray(q2t),
                  wr2b, bc2t, wf1b, bf1t, wf2b, bf2t)
    out2d = pl.pallas_call(
        _conv2_head_kernel,
        out_shape=jax.ShapeDtypeStruct((n_hi, 64), f32),
        grid=grid,
        in_specs=[
            pl.BlockSpec((te, 2), lambda t: (t, 0)),
            pl.BlockSpec((te, 8), lambda t: (t, 0)),
        ] + [_full(a) for a in conv2_args[2:]],
        out_specs=pl.BlockSpec((n_hi, 64), lambda t: (0, 0)),
        scratch_shapes=[pltpu.VMEM((n_hi, 32 * 32), f32)],
        compiler_params=cparams,
    )(*conv2_args)
    return out2d.reshape(n, 2)


# speedup vs baseline: 1.8331x; 1.8331x over previous
import numpy as np
import jax
import jax.numpy as jnp
from jax import lax
from jax.experimental import pallas as pl
from jax.experimental.pallas import tpu as pltpu


def _edge_tile(e_pad):
    for te in (512, 256, 128):
        if e_pad % te == 0:
            return te
    return e_pad


def _conv1_extract_kernel(ea_ref, s_ref, m_ref, xb_ref,
                          w1a_ref, b1a_ref, w1b_ref, b1b_ref,
                          wr1_ref, bc1_ref, bd_ref,
                          h1_ref, idx_ref, acc_ref):
    t = pl.program_id(0)
    f32 = jnp.float32

    @pl.when(t == 0)
    def _init():
        acc_ref[...] = jnp.zeros_like(acc_ref)

    ea = ea_ref[...]
    w1a = w1a_ref[...]
    hid = jnp.maximum(ea[:, 0:1] * w1a[0:1, :] + ea[:, 1:2] * w1a[1:2, :]
                      + b1a_ref[...], 0.0)
    z = jnp.dot(hid, w1b_ref[...], preferred_element_type=f32) + b1b_ref[...]

    s = s_ref[...]
    m = m_ref[...]

    xgb = jnp.dot(s, xb_ref[...], preferred_element_type=f32)
    xg = xgb[:, 0:2]
    msg = xg[:, 0:1] * z[:, 0:32] + xg[:, 1:2] * z[:, 32:64]
    acc_ref[...] += jnp.dot(m, msg, preferred_element_type=f32)

    idx_ref[...] = xgb + lax.dot_general(m, bd_ref[...],
                                         (((0,), (0,)), ((), ())),
                                         preferred_element_type=f32)

    @pl.when(t == pl.num_programs(0) - 1)
    def _finalize():
        x = xb_ref[...][:, 0:2]
        wr = wr1_ref[...]
        root = x[:, 0:1] * wr[0:1, :] + x[:, 1:2] * wr[1:2, :]
        h1_ref[...] = jnp.maximum(acc_ref[...] + root + bc1_ref[...], 0.0)


def _conv2_head_kernel(ea_ref, idx_ref, hrs_ref,
                       w2a_ref, b2a_ref, w2b_ref, b2b_ref,
                       r2_ref, q2_ref, q2t_ref,
                       wr2b_ref, bc2t_ref, wf1b_ref, bf1t_ref,
                       wf2b_ref, bf2t_ref,
                       out_ref, acc_ref):
    t = pl.program_id(0)
    f32 = jnp.float32
    n_hi = acc_ref.shape[0]
    te = ea_ref.shape[0]

    @pl.when(t == 0)
    def _init():
        acc_ref[...] = jnp.zeros_like(acc_ref)

    ea = ea_ref[...]
    w2a = w2a_ref[...]
    hid = jnp.maximum(ea[:, 0:1] * w2a[0:1, :] + ea[:, 1:2] * w2a[1:2, :]
                      + b2a_ref[...], 0.0)
    z = jnp.dot(hid, w2b_ref[...], preferred_element_type=f32) + b2b_ref[...]

    idx = idx_ref[...]
    shi = jnp.round(idx[:, 2:3] * (1.0 / 32.0))
    slo = jnp.round(idx[:, 3:4])
    w = idx[:, 6:7]
    winv = 1.0 / jnp.maximum(w, 1e-30)
    dhi = jnp.round(idx[:, 4:5] * winv * (1.0 / 32.0))
    dlo = jnp.round(idx[:, 5:6] * winv)

    ihi = lax.broadcasted_iota(jnp.int32, (te, n_hi), 1).astype(f32)
    ilo = lax.broadcasted_iota(jnp.int32, (te, 32), 1).astype(f32)
    oh_shi = (shi == ihi).astype(f32)
    oh_slo = (slo == ilo).astype(f32)
    oh_dhi = (dhi == ihi).astype(f32)
    oh_dlo = (dlo == ilo).astype(f32)

    r2 = r2_ref[...]
    q2 = q2_ref[...]
    hrs = hrs_ref[...]

    hb = jnp.dot(oh_shi, hrs, preferred_element_type=f32)
    rep_slo = jnp.dot(oh_slo, r2, preferred_element_type=f32)
    hg = jnp.dot(hb * rep_slo, q2, preferred_element_type=f32)

    hg_rep = jnp.dot(hg, r2, preferred_element_type=f32)
    msg = jnp.dot(hg_rep * z, q2, preferred_element_type=f32)

    msg_t = jnp.dot(w * msg, q2t_ref[...], preferred_element_type=f32)
    rep_dlo = jnp.dot(oh_dlo, r2, preferred_element_type=f32)
    acc_ref[...] += lax.dot_general(oh_dhi, rep_dlo * msg_t,
                                    (((0,), (0,)), ((), ())),
                                    preferred_element_type=f32)

    @pl.when(t == pl.num_programs(0) - 1)
    def _finalize():
        hrs_f = hrs_ref[...]
        h2 = jnp.maximum(acc_ref[...]
                         + jnp.dot(hrs_f, wr2b_ref[...], preferred_element_type=f32)
                         + bc2t_ref[...], 0.0)
        h3 = jnp.maximum(jnp.dot(h2, wf1b_ref[...], preferred_element_type=f32)
                         + bf1t_ref[...], 0.0)
        out_ref[...] = (jnp.dot(h3, wf2b_ref[...], preferred_element_type=f32)
                        + bf2t_ref[...])


def _full(arr):
    nd = arr.ndim
    return pl.BlockSpec(arr.shape, lambda t, _n=nd: (0,) * _n)


def kernel(x, edge_attr_pad, S, M,
           w1a, b1a, w1b, b1b, w2a, b2a, w2b, b2b,
           wr1, bc1, wr2, bc2, wfc1, bfc1, wfc2, bfc2, r2, q2):
    f32 = jnp.float32
    n = x.shape[0]
    e_pad = edge_attr_pad.shape[0]
    te = _edge_tile(e_pad)
    grid = (e_pad // te,)
    cparams = pltpu.CompilerParams(dimension_semantics=("arbitrary",))

    ar = np.arange(n)
    hi32 = (32 * (ar // 32)).astype(np.float32)
    lo = (ar % 32).astype(np.float32)
    xcols = np.zeros((n, 6), np.float32)
    xcols[:, 0] = hi32
    xcols[:, 1] = lo
    bd = np.zeros((n, 8), np.float32)
    bd[:, 4] = hi32
    bd[:, 5] = lo
    bd[:, 6] = 1.0
    jj = np.arange(32 * 32)
    q2t = (jj[None, :] % 32 == np.arange(32)[:, None]).astype(np.float32)

    xb = jnp.concatenate([x, jnp.asarray(xcols)], axis=1)
    conv1_args = (edge_attr_pad, S, M, xb, w1a, b1a, w1b, b1b, wr1, bc1,
                  jnp.asarray(bd))
    h1, idx = pl.pallas_call(
        _conv1_extract_kernel,
        out_shape=[jax.ShapeDtypeStruct((n, 32), f32),
                   jax.ShapeDtypeStruct((e_pad, 8), f32)],
        grid=grid,
        in_specs=[
            pl.BlockSpec((te, 2), lambda t: (t, 0)),
            pl.BlockSpec((te, n), lambda t: (t, 0)),
            pl.BlockSpec((n, te), lambda t: (0, t)),
        ] + [_full(a) for a in conv1_args[3:]],
        out_specs=[pl.BlockSpec((n, 32), lambda t: (0, 0)),
                   pl.BlockSpec((te, 8), lambda t: (t, 0))],
        scratch_shapes=[pltpu.VMEM((n, 32), f32)],
        compiler_params=cparams,
    )(*conv1_args)

    n_hi = n // 32
    h1_rs = h1.reshape(n_hi, 32 * 32)
    eye32 = jnp.eye(32, dtype=f32)
    wr2b = jnp.kron(eye32, wr2)
    wf1b = jnp.kron(eye32, wfc1)
    wf2b = jnp.kron(eye32, wfc2)
    bc2t = jnp.tile(bc2, (1, 32))
    bf1t = jnp.tile(bfc1, (1, 32))
    bf2t = jnp.tile(bfc2, (1, 32))

    conv2_args = (edge_attr_pad, idx, h1_rs, w2a, b2a, w2b, b2b,
                  r2, q2, jnp.asarray(q2t),
                  wr2b, bc2t, wf1b, bf1t, wf2b, bf2t)
    out2d = pl.pallas_call(
        _conv2_head_kernel,
        out_shape=jax.ShapeDtypeStruct((n_hi, 64), f32),
        grid=grid,
        in_specs=[
            pl.BlockSpec((te, 2), lambda t: (t, 0)),
            pl.BlockSpec((te, 8), lambda t: (t, 0)),
        ] + [_full(a) for a in conv2_args[2:]],
        out_specs=pl.BlockSpec((n_hi, 64), lambda t: (0, 0)),
        scratch_shapes=[pltpu.VMEM((n_hi, 32 * 32), f32)],
        compiler_params=cparams,
    )(*conv2_args)
    return out2d.reshape(n, 2)
